# baseline (speedup 1.0000x reference)
import functools

import jax
import jax.numpy as jnp
from jax import lax
from jax.experimental import pallas as pl
from jax.experimental.pallas import tpu as pltpu
from jax.experimental.pallas import tpu_sc as plsc

_NC = 2
_NS = 16
_NW = _NC * _NS


def _dist_body(zf_ref, cb_ref, zsq_ref, wsq_ref, idx_ref, dsum_ref, *, T, K):
    zb = zf_ref[...]
    cb = cb_ref[...]
    zw = lax.dot_general(zb, cb, (((1,), (1,)), ((), ())),
                         preferred_element_type=jnp.float32)
    zsq = jnp.swapaxes(zsq_ref[0], 0, 1)
    wsq = wsq_ref[...]
    d2 = (zsq + wsq) - 2.0 * zw
    dist = jnp.sqrt(jnp.maximum(d2, 0.0))
    rowmin = jnp.min(dist, axis=1, keepdims=True)
    kiota = lax.broadcasted_iota(jnp.int32, (T, K), 1)
    idx = jnp.min(jnp.where(dist == rowmin, kiota, K), axis=1)
    idx_ref[0, 0, :] = idx.astype(jnp.int32)

    @pl.when(pl.program_id(0) == 0)
    def _():
        dsum_ref[0, 0] = 0.0

    dsum_ref[0, 0] += jnp.sum(rowmin * rowmin)


def _distances_argmin(z_flat, codebook, T):
    N, D = z_flat.shape
    K = codebook.shape[0]
    nb = N // T
    z_sq = jnp.sum(z_flat ** 2, axis=1)
    w_sq = jnp.sum(codebook ** 2, axis=1)
    idx3, dsum = pl.pallas_call(
        functools.partial(_dist_body, T=T, K=K),
        grid=(nb,),
        in_specs=[
            pl.BlockSpec((T, D), lambda i: (i, 0)),
            pl.BlockSpec((K, D), lambda i: (0, 0)),
            pl.BlockSpec((1, 1, T), lambda i: (i, 0, 0)),
            pl.BlockSpec((1, K), lambda i: (0, 0)),
        ],
        out_specs=[
            pl.BlockSpec((1, 1, T), lambda i: (i, 0, 0)),
            pl.BlockSpec(memory_space=pltpu.SMEM),
        ],
        out_shape=[
            jax.ShapeDtypeStruct((nb, 1, T), jnp.int32),
            jax.ShapeDtypeStruct((1, 1), jnp.float32),
        ],
    )(z_flat, codebook, z_sq.reshape(nb, 1, T), w_sq.reshape(1, K))
    return idx3.reshape(N), dsum


def _sc_body(cb_hbm, idx_hbm, ones_hbm, zeros_hbm, q_hbm, cnt_hbm,
             idx_v, rows_v, ones_v, shared_cnt, sem, *, BPW, CH, CW):
    c = lax.axis_index("c")
    s = lax.axis_index("s")
    w = s * _NC + c
    nch = BPW // CH

    @pl.when(s == 0)
    def _():
        pltpu.sync_copy(zeros_hbm, shared_cnt)

    pltpu.sync_copy(idx_hbm.at[w], idx_v)
    pltpu.sync_copy(ones_hbm, ones_v)
    plsc.subcore_barrier()
    for j in range(nch):
        pltpu.async_copy(cb_hbm.at[idx_v.at[j]],
                         rows_v.at[pl.ds(j * CH, CH)], sem).wait()
        pltpu.sync_copy(ones_v, shared_cnt.at[idx_v.at[j]], add=True)
    pltpu.sync_copy(rows_v, q_hbm.at[pl.ds(w * BPW, BPW)])
    plsc.subcore_barrier()

    @pl.when(s == 0)
    def _():
        pltpu.sync_copy(shared_cnt, cnt_hbm.at[c])


def _sc_gather_counts(codebook, idx, N, K):
    D = codebook.shape[1]
    CW = 128
    BPW = N // _NW
    CH = 128
    nch = BPW // CH
    idx3 = idx.reshape(_NW, nch, CH)
    ones = jnp.ones((CH, CW), jnp.int32)
    zeros = jnp.zeros((K, CW), jnp.int32)
    mesh = plsc.VectorSubcoreMesh(core_axis_name="c", subcore_axis_name="s",
                                  num_cores=_NC, num_subcores=_NS)
    fn = pl.kernel(
        functools.partial(_sc_body, BPW=BPW, CH=CH, CW=CW),
        out_type=[
            jax.ShapeDtypeStruct((N, D), jnp.float32),
            jax.ShapeDtypeStruct((_NC, K, CW), jnp.int32),
        ],
        mesh=mesh,
        compiler_params=pltpu.CompilerParams(use_tc_tiling_on_sc=False),
        scratch_types=[
            pltpu.VMEM((nch, CH), jnp.int32),
            pltpu.VMEM((BPW, D), jnp.float32),
            pltpu.VMEM((CH, CW), jnp.int32),
            pltpu.VMEM_SHARED((K, CW), jnp.int32),
            pltpu.SemaphoreType.DMA,
        ],
    )
    q_flat, cnt = fn(codebook, idx3, ones, zeros)
    return q_flat, cnt


def _final_body(cnt_ref, dsum_ref, vq_ref, perp_ref, act_ref, *, K, N, ND):
    cnt = cnt_ref[...]
    c01 = cnt[0:K, 0:1] + cnt[K:2 * K, 0:1]
    counts = c01.astype(jnp.float32)
    probs = counts * (1.0 / N)
    ent = probs * jnp.log(probs + 1e-10)
    perp_ref[0, 0] = jnp.exp(-jnp.sum(ent))
    act_ref[0, 0] = jnp.sum((c01 > 0).astype(jnp.int32))
    m = dsum_ref[0, 0] * (1.0 / ND)
    vq_ref[0, 0] = m + 0.25 * m


def _finalize(cnt, dsum, N, K, ND):
    cnt2 = cnt.reshape(_NC * K, cnt.shape[-1])
    vq, perp, act = pl.pallas_call(
        functools.partial(_final_body, K=K, N=N, ND=ND),
        in_specs=[
            pl.BlockSpec(cnt2.shape, lambda: (0, 0)),
            pl.BlockSpec(memory_space=pltpu.SMEM),
        ],
        out_specs=[
            pl.BlockSpec(memory_space=pltpu.SMEM),
            pl.BlockSpec(memory_space=pltpu.SMEM),
            pl.BlockSpec(memory_space=pltpu.SMEM),
        ],
        out_shape=[
            jax.ShapeDtypeStruct((1, 1), jnp.float32),
            jax.ShapeDtypeStruct((1, 1), jnp.float32),
            jax.ShapeDtypeStruct((1, 1), jnp.int32),
        ],
    )(cnt2, dsum)
    return vq[0, 0], perp[0, 0], act[0, 0]


def kernel(z, codebook):
    orig_shape = z.shape
    B, C = z.shape[0], z.shape[1]
    K, D = codebook.shape
    z_flat = z.reshape(B, C, -1).transpose(0, 2, 1).reshape(-1, C)
    N = z_flat.shape[0]

    idx, dsum = _distances_argmin(z_flat, codebook, T=1024)
    q_flat, cnt = _sc_gather_counts(codebook, idx, N, K)
    vq_loss, perplexity, active_codes = _finalize(cnt, dsum, N, K, N * D)

    quantized = q_flat.reshape(B, -1, C).transpose(0, 2, 1).reshape(orig_shape)
    return (quantized, idx, vq_loss, perplexity, active_codes)


# speedup vs baseline: 1.1552x; 1.1552x over previous
import functools

import jax
import jax.numpy as jnp
from jax import lax
from jax.experimental import pallas as pl
from jax.experimental.pallas import tpu as pltpu
from jax.experimental.pallas import tpu_sc as plsc

_NC = 2
_NS = 16
_NW = _NC * _NS


def _dist_body(zf_ref, cb_ref, zsq_ref, wsq_ref, idx_ref, dsum_ref, *, T, K):
    zb = zf_ref[...]
    cb = cb_ref[...]
    zw = lax.dot_general(zb, cb, (((1,), (1,)), ((), ())),
                         preferred_element_type=jnp.float32)
    zsq = jnp.swapaxes(zsq_ref[0], 0, 1)
    wsq = wsq_ref[...]
    d2 = (zsq + wsq) - 2.0 * zw
    dist = jnp.sqrt(jnp.maximum(d2, 0.0))
    rowmin = jnp.min(dist, axis=1, keepdims=True)
    kiota = lax.broadcasted_iota(jnp.int32, (T, K), 1)
    idx = jnp.min(jnp.where(dist == rowmin, kiota, K), axis=1)
    idx_ref[0, 0, :] = idx.astype(jnp.int32)

    @pl.when(pl.program_id(0) == 0)
    def _():
        dsum_ref[0, 0] = 0.0

    dsum_ref[0, 0] += jnp.sum(rowmin * rowmin)


def _distances_argmin(z_flat, codebook, T):
    N, D = z_flat.shape
    K = codebook.shape[0]
    nb = N // T
    z_sq = jnp.sum(z_flat ** 2, axis=1)
    w_sq = jnp.sum(codebook ** 2, axis=1)
    idx3, dsum = pl.pallas_call(
        functools.partial(_dist_body, T=T, K=K),
        grid=(nb,),
        in_specs=[
            pl.BlockSpec((T, D), lambda i: (i, 0)),
            pl.BlockSpec((K, D), lambda i: (0, 0)),
            pl.BlockSpec((1, 1, T), lambda i: (i, 0, 0)),
            pl.BlockSpec((1, K), lambda i: (0, 0)),
        ],
        out_specs=[
            pl.BlockSpec((1, 1, T), lambda i: (i, 0, 0)),
            pl.BlockSpec(memory_space=pltpu.SMEM),
        ],
        out_shape=[
            jax.ShapeDtypeStruct((nb, 1, T), jnp.int32),
            jax.ShapeDtypeStruct((1, 1), jnp.float32),
        ],
    )(z_flat, codebook, z_sq.reshape(nb, 1, T), w_sq.reshape(1, K))
    return idx3.reshape(N), dsum


def _sc_body(cb_hbm, idx_hbm, ones_hbm, zeros_hbm, q_hbm, cnt_hbm,
             idx_v, rows_v, ones_v, shared_cnt, sem, *, BPW, CH):
    c = lax.axis_index("c")
    s = lax.axis_index("s")
    w = s * _NC + c
    nch = BPW // CH

    @pl.when(s == 0)
    def _():
        pltpu.sync_copy(zeros_hbm, shared_cnt)

    pltpu.sync_copy(idx_hbm.at[w], idx_v)
    pltpu.sync_copy(ones_hbm, ones_v)
    plsc.subcore_barrier()
    for j in range(nch):
        pltpu.async_copy(cb_hbm.at[idx_v.at[j]],
                         rows_v.at[pl.ds(j * CH, CH)], sem).wait()
        pltpu.sync_copy(ones_v, shared_cnt.at[idx_v.at[j]], add=True)
    pltpu.sync_copy(rows_v, q_hbm.at[pl.ds(w * BPW, BPW)])
    plsc.subcore_barrier()

    @pl.when(s == 0)
    def _():
        pltpu.sync_copy(shared_cnt, cnt_hbm.at[c])


def _sc_gather_counts(codebook, idx, N, K):
    D = codebook.shape[1]
    DP = 128
    cb_pad = jnp.pad(codebook, ((0, 0), (0, DP - D)))
    BPW = N // _NW
    CH = 128
    nch = BPW // CH
    idx3 = idx.reshape(_NW, nch, CH)
    ones = jnp.ones((CH, DP), jnp.int32)
    zeros = jnp.zeros((K, DP), jnp.int32)
    mesh = plsc.VectorSubcoreMesh(core_axis_name="c", subcore_axis_name="s",
                                  num_cores=_NC, num_subcores=_NS)
    fn = pl.kernel(
        functools.partial(_sc_body, BPW=BPW, CH=CH),
        out_type=[
            jax.ShapeDtypeStruct((N, DP), jnp.float32),
            jax.ShapeDtypeStruct((_NC, K, DP), jnp.int32),
        ],
        mesh=mesh,
        scratch_types=[
            pltpu.VMEM((nch, CH), jnp.int32),
            pltpu.VMEM((BPW, DP), jnp.float32),
            pltpu.VMEM((CH, DP), jnp.int32),
            pltpu.VMEM_SHARED((K, DP), jnp.int32),
            pltpu.SemaphoreType.DMA,
        ],
    )
    q_pad, cnt = fn(cb_pad, idx3, ones, zeros)
    return q_pad[:, :D], cnt


def _final_body(cnt_ref, dsum_ref, vq_ref, perp_ref, act_ref, *, K, N, ND):
    cnt = cnt_ref[...]
    c01 = cnt[0:K, 0:1] + cnt[K:2 * K, 0:1]
    counts = c01.astype(jnp.float32)
    probs = counts * (1.0 / N)
    ent = probs * jnp.log(probs + 1e-10)
    perp_ref[0, 0] = jnp.exp(-jnp.sum(ent))
    act_ref[0, 0] = jnp.sum((c01 > 0).astype(jnp.int32))
    m = dsum_ref[0, 0] * (1.0 / ND)
    vq_ref[0, 0] = m + 0.25 * m


def _finalize(cnt, dsum, N, K, ND):
    cnt2 = cnt.reshape(_NC * K, cnt.shape[-1])
    vq, perp, act = pl.pallas_call(
        functools.partial(_final_body, K=K, N=N, ND=ND),
        in_specs=[
            pl.BlockSpec(cnt2.shape, lambda: (0, 0)),
            pl.BlockSpec(memory_space=pltpu.SMEM),
        ],
        out_specs=[
            pl.BlockSpec(memory_space=pltpu.SMEM),
            pl.BlockSpec(memory_space=pltpu.SMEM),
            pl.BlockSpec(memory_space=pltpu.SMEM),
        ],
        out_shape=[
            jax.ShapeDtypeStruct((1, 1), jnp.float32),
            jax.ShapeDtypeStruct((1, 1), jnp.float32),
            jax.ShapeDtypeStruct((1, 1), jnp.int32),
        ],
    )(cnt2, dsum)
    return vq[0, 0], perp[0, 0], act[0, 0]


def kernel(z, codebook):
    orig_shape = z.shape
    B, C = z.shape[0], z.shape[1]
    K, D = codebook.shape
    z_flat = z.reshape(B, C, -1).transpose(0, 2, 1).reshape(-1, C)
    N = z_flat.shape[0]

    idx, dsum = _distances_argmin(z_flat, codebook, T=2048)
    q_flat, cnt = _sc_gather_counts(codebook, idx, N, K)
    vq_loss, perplexity, active_codes = _finalize(cnt, dsum, N, K, N * D)

    quantized = q_flat.reshape(B, -1, C).transpose(0, 2, 1).reshape(orig_shape)
    return (quantized, idx, vq_loss, perplexity, active_codes)


# speedup vs baseline: 1.1638x; 1.0074x over previous
import functools

import jax
import jax.numpy as jnp
from jax import lax
from jax.experimental import pallas as pl
from jax.experimental.pallas import tpu as pltpu
from jax.experimental.pallas import tpu_sc as plsc

_NC = 2
_NS = 16
_NW = _NC * _NS


def _dist_body(zf_ref, cb_ref, zsq_ref, wsq_ref, idx_ref, dsum_ref, *, T, K):
    zb = zf_ref[...]
    cb = cb_ref[...]
    zw = lax.dot_general(zb, cb, (((1,), (1,)), ((), ())),
                         preferred_element_type=jnp.float32)
    zsq = jnp.swapaxes(zsq_ref[0], 0, 1)
    wsq = wsq_ref[...]
    d2 = (zsq + wsq) - 2.0 * zw
    dist = jnp.sqrt(jnp.maximum(d2, 0.0))
    rowmin = jnp.min(dist, axis=1, keepdims=True)
    kiota = lax.broadcasted_iota(jnp.int32, (T, K), 1)
    idx = jnp.min(jnp.where(dist == rowmin, kiota, K), axis=1)
    idx_ref[0, 0, :] = idx.astype(jnp.int32)

    @pl.when(pl.program_id(0) == 0)
    def _():
        dsum_ref[0, 0] = 0.0

    dsum_ref[0, 0] += jnp.sum(rowmin * rowmin)


def _distances_argmin(z_flat, codebook, T):
    N, D = z_flat.shape
    K = codebook.shape[0]
    nb = N // T
    z_sq = jnp.sum(z_flat ** 2, axis=1)
    w_sq = jnp.sum(codebook ** 2, axis=1)
    idx3, dsum = pl.pallas_call(
        functools.partial(_dist_body, T=T, K=K),
        grid=(nb,),
        in_specs=[
            pl.BlockSpec((T, D), lambda i: (i, 0)),
            pl.BlockSpec((K, D), lambda i: (0, 0)),
            pl.BlockSpec((1, 1, T), lambda i: (i, 0, 0)),
            pl.BlockSpec((1, K), lambda i: (0, 0)),
        ],
        out_specs=[
            pl.BlockSpec((1, 1, T), lambda i: (i, 0, 0)),
            pl.BlockSpec(memory_space=pltpu.SMEM),
        ],
        out_shape=[
            jax.ShapeDtypeStruct((nb, 1, T), jnp.int32),
            jax.ShapeDtypeStruct((1, 1), jnp.float32),
        ],
    )(z_flat, codebook, z_sq.reshape(nb, 1, T), w_sq.reshape(1, K))
    return idx3.reshape(N), dsum


def _sc_body(cb_hbm, idx_hbm, ones_hbm, zeros_hbm, q_hbm, cnt_hbm,
             idx_v, rows_v, ones_v, shared_cnt, sem, *, BPW, CH):
    c = lax.axis_index("c")
    s = lax.axis_index("s")
    w = s * _NC + c
    nch = BPW // CH

    @pl.when(s == 0)
    def _():
        pltpu.sync_copy(zeros_hbm, shared_cnt)

    pltpu.sync_copy(idx_hbm.at[w], idx_v)
    pltpu.sync_copy(ones_hbm, ones_v)
    plsc.subcore_barrier()
    for j in range(nch):
        pltpu.async_copy(cb_hbm.at[idx_v.at[j]],
                         rows_v.at[pl.ds(j * CH, CH)], sem).wait()
        pltpu.sync_copy(ones_v, shared_cnt.at[idx_v.at[j]], add=True)
    pltpu.sync_copy(rows_v, q_hbm.at[pl.ds(w * BPW, BPW)])
    plsc.subcore_barrier()

    @pl.when(s == 0)
    def _():
        pltpu.sync_copy(shared_cnt, cnt_hbm.at[c])


def _sc_gather_counts(codebook, idx, N, K):
    D = codebook.shape[1]
    DP = 128
    cb_pad = jnp.pad(codebook, ((0, 0), (0, DP - D)))
    BPW = N // _NW
    CH = 128
    nch = BPW // CH
    idx3 = idx.reshape(_NW, nch, CH)
    ones = jnp.ones((CH, DP), jnp.int32)
    zeros = jnp.zeros((K, DP), jnp.int32)
    mesh = plsc.VectorSubcoreMesh(core_axis_name="c", subcore_axis_name="s",
                                  num_cores=_NC, num_subcores=_NS)
    fn = pl.kernel(
        functools.partial(_sc_body, BPW=BPW, CH=CH),
        out_type=[
            jax.ShapeDtypeStruct((N, DP), jnp.float32),
            jax.ShapeDtypeStruct((_NC, K, DP), jnp.int32),
        ],
        mesh=mesh,
        scratch_types=[
            pltpu.VMEM((nch, CH), jnp.int32),
            pltpu.VMEM((BPW, DP), jnp.float32),
            pltpu.VMEM((CH, DP), jnp.int32),
            pltpu.VMEM_SHARED((K, DP), jnp.int32),
            pltpu.SemaphoreType.DMA,
        ],
    )
    q_pad, cnt = fn(cb_pad, idx3, ones, zeros)
    return q_pad[:, :D], cnt


def _final_body(cnt_ref, dsum_ref, vq_ref, perp_ref, act_ref, *, K, N, ND):
    cnt = cnt_ref[...]
    c01 = cnt[0:K, 0:1] + cnt[K:2 * K, 0:1]
    counts = c01.astype(jnp.float32)
    probs = counts * (1.0 / N)
    ent = probs * jnp.log(probs + 1e-10)
    perp_ref[0, 0] = jnp.exp(-jnp.sum(ent))
    act_ref[0, 0] = jnp.sum((c01 > 0).astype(jnp.int32))
    m = dsum_ref[0, 0] * (1.0 / ND)
    vq_ref[0, 0] = m + 0.25 * m


def _finalize(cnt, dsum, N, K, ND):
    cnt2 = cnt.reshape(_NC * K, cnt.shape[-1])
    vq, perp, act = pl.pallas_call(
        functools.partial(_final_body, K=K, N=N, ND=ND),
        in_specs=[
            pl.BlockSpec(cnt2.shape, lambda: (0, 0)),
            pl.BlockSpec(memory_space=pltpu.SMEM),
        ],
        out_specs=[
            pl.BlockSpec(memory_space=pltpu.SMEM),
            pl.BlockSpec(memory_space=pltpu.SMEM),
            pl.BlockSpec(memory_space=pltpu.SMEM),
        ],
        out_shape=[
            jax.ShapeDtypeStruct((1, 1), jnp.float32),
            jax.ShapeDtypeStruct((1, 1), jnp.float32),
            jax.ShapeDtypeStruct((1, 1), jnp.int32),
        ],
    )(cnt2, dsum)
    return vq[0, 0], perp[0, 0], act[0, 0]


def kernel(z, codebook):
    orig_shape = z.shape
    B, C = z.shape[0], z.shape[1]
    K, D = codebook.shape
    z_flat = z.reshape(B, C, -1).transpose(0, 2, 1).reshape(-1, C)
    N = z_flat.shape[0]

    idx, dsum = _distances_argmin(z_flat, codebook, T=4096)
    q_flat, cnt = _sc_gather_counts(codebook, idx, N, K)
    vq_loss, perplexity, active_codes = _finalize(cnt, dsum, N, K, N * D)

    quantized = q_flat.reshape(B, -1, C).transpose(0, 2, 1).reshape(orig_shape)
    return (quantized, idx, vq_loss, perplexity, active_codes)


# speedup vs baseline: 1.2328x; 1.0593x over previous
import functools

import jax
import jax.numpy as jnp
from jax import lax
from jax.experimental import pallas as pl
from jax.experimental.pallas import tpu as pltpu
from jax.experimental.pallas import tpu_sc as plsc

_NC = 2
_NS = 16
_NW = _NC * _NS


def _dist_body(zf_ref, cb_ref, zsq_ref, wsq_ref, idx_ref, dsum_ref, *, T, K):
    zb = zf_ref[...]
    cb = cb_ref[...]
    zw = lax.dot_general(zb, cb, (((1,), (1,)), ((), ())),
                         preferred_element_type=jnp.float32)
    zsq = jnp.swapaxes(zsq_ref[0], 0, 1)
    wsq = wsq_ref[...]
    d2 = (zsq + wsq) - 2.0 * zw
    dist = jnp.sqrt(jnp.maximum(d2, 0.0))
    rowmin = jnp.min(dist, axis=1, keepdims=True)
    kiota = lax.broadcasted_iota(jnp.int32, (T, K), 1)
    idx = jnp.min(jnp.where(dist == rowmin, kiota, K), axis=1)
    idx_ref[0, 0, :] = idx.astype(jnp.int32)

    @pl.when(pl.program_id(0) == 0)
    def _():
        dsum_ref[0, 0] = 0.0

    dsum_ref[0, 0] += jnp.sum(rowmin * rowmin)


def _distances_argmin(z_flat, codebook, T):
    N, D = z_flat.shape
    K = codebook.shape[0]
    nb = N // T
    z_sq = jnp.sum(z_flat ** 2, axis=1)
    w_sq = jnp.sum(codebook ** 2, axis=1)
    idx3, dsum = pl.pallas_call(
        functools.partial(_dist_body, T=T, K=K),
        grid=(nb,),
        in_specs=[
            pl.BlockSpec((T, D), lambda i: (i, 0)),
            pl.BlockSpec((K, D), lambda i: (0, 0)),
            pl.BlockSpec((1, 1, T), lambda i: (i, 0, 0)),
            pl.BlockSpec((1, K), lambda i: (0, 0)),
        ],
        out_specs=[
            pl.BlockSpec((1, 1, T), lambda i: (i, 0, 0)),
            pl.BlockSpec(memory_space=pltpu.SMEM),
        ],
        out_shape=[
            jax.ShapeDtypeStruct((nb, 1, T), jnp.int32),
            jax.ShapeDtypeStruct((1, 1), jnp.float32),
        ],
    )(z_flat, codebook, z_sq.reshape(nb, 1, T), w_sq.reshape(1, K))
    return idx3.reshape(N), dsum


def _sc_body(cb_hbm, idx_hbm, zeros_hbm, q_hbm, cnt_hbm,
             idx_v, rows_v, cnt_v, sem, *, BPW, CH, K):
    c = lax.axis_index("c")
    s = lax.axis_index("s")
    w = s * _NC + c
    nch = BPW // CH

    pltpu.sync_copy(idx_hbm.at[w], idx_v)
    pltpu.sync_copy(zeros_hbm, cnt_v)
    for j in range(nch):
        pltpu.async_copy(cb_hbm.at[idx_v.at[j]],
                         rows_v.at[pl.ds(j * CH, CH)], sem).wait()
        for o in range(0, CH, 16):
            ivec = idx_v[j, pl.ds(o, 16)]
            plsc.addupdate_scatter(cnt_v, [ivec], jnp.ones((16,), jnp.int32))
    pltpu.sync_copy(rows_v, q_hbm.at[pl.ds(w * BPW, BPW)])
    pltpu.sync_copy(cnt_v, cnt_hbm.at[w])


def _sc_gather_counts(codebook, idx, N, K):
    D = codebook.shape[1]
    DP = 128
    cb_pad = jnp.pad(codebook, ((0, 0), (0, DP - D)))
    BPW = N // _NW
    CH = 128
    nch = BPW // CH
    idx3 = idx.reshape(_NW, nch, CH)
    zeros = jnp.zeros((K,), jnp.int32)
    mesh = plsc.VectorSubcoreMesh(core_axis_name="c", subcore_axis_name="s",
                                  num_cores=_NC, num_subcores=_NS)
    fn = pl.kernel(
        functools.partial(_sc_body, BPW=BPW, CH=CH, K=K),
        out_type=[
            jax.ShapeDtypeStruct((N, DP), jnp.float32),
            jax.ShapeDtypeStruct((_NW, K), jnp.int32),
        ],
        mesh=mesh,
        compiler_params=pltpu.CompilerParams(needs_layout_passes=False),
        scratch_types=[
            pltpu.VMEM((nch, CH), jnp.int32),
            pltpu.VMEM((BPW, DP), jnp.float32),
            pltpu.VMEM((K,), jnp.int32),
            pltpu.SemaphoreType.DMA,
        ],
    )
    q_pad, cnt = fn(cb_pad, idx3, zeros)
    return q_pad[:, :D], cnt


def _final_body(cnt_ref, dsum_ref, vq_ref, perp_ref, act_ref, *, K, N, ND):
    cnt = cnt_ref[...]
    c01 = jnp.sum(cnt, axis=0, keepdims=True)
    counts = c01.astype(jnp.float32)
    probs = counts * (1.0 / N)
    ent = probs * jnp.log(probs + 1e-10)
    perp_ref[0, 0] = jnp.exp(-jnp.sum(ent))
    act_ref[0, 0] = jnp.sum((c01 > 0).astype(jnp.int32))
    m = dsum_ref[0, 0] * (1.0 / ND)
    vq_ref[0, 0] = m + 0.25 * m


def _finalize(cnt, dsum, N, K, ND):
    cnt2 = cnt
    vq, perp, act = pl.pallas_call(
        functools.partial(_final_body, K=K, N=N, ND=ND),
        in_specs=[
            pl.BlockSpec(cnt2.shape, lambda: (0, 0)),
            pl.BlockSpec(memory_space=pltpu.SMEM),
        ],
        out_specs=[
            pl.BlockSpec(memory_space=pltpu.SMEM),
            pl.BlockSpec(memory_space=pltpu.SMEM),
            pl.BlockSpec(memory_space=pltpu.SMEM),
        ],
        out_shape=[
            jax.ShapeDtypeStruct((1, 1), jnp.float32),
            jax.ShapeDtypeStruct((1, 1), jnp.float32),
            jax.ShapeDtypeStruct((1, 1), jnp.int32),
        ],
    )(cnt2, dsum)
    return vq[0, 0], perp[0, 0], act[0, 0]


def kernel(z, codebook):
    orig_shape = z.shape
    B, C = z.shape[0], z.shape[1]
    K, D = codebook.shape
    z_flat = z.reshape(B, C, -1).transpose(0, 2, 1).reshape(-1, C)
    N = z_flat.shape[0]

    idx, dsum = _distances_argmin(z_flat, codebook, T=4096)
    q_flat, cnt = _sc_gather_counts(codebook, idx, N, K)
    vq_loss, perplexity, active_codes = _finalize(cnt, dsum, N, K, N * D)

    quantized = q_flat.reshape(B, -1, C).transpose(0, 2, 1).reshape(orig_shape)
    return (quantized, idx, vq_loss, perplexity, active_codes)
